# Initial kernel scaffold; baseline (speedup 1.0000x reference)
import jax
import jax.numpy as jnp
from jax.experimental import pallas as pl


def kernel(z_e_x, W):
    raise NotImplementedError("write your pallas kernel here")


# speedup vs baseline: 1.0180x; 1.0180x over previous
import jax
import jax.numpy as jnp
from jax.experimental import pallas as pl
from jax.experimental.pallas import tpu as pltpu

N_CODES = 8192
N_TOK = 8192
D = 32
SUB = 256
BLOCK = 2048
STEPS_PER_BLOCK = BLOCK // SUB


def _bf16_round(x):
    return x.astype(jnp.bfloat16).astype(jnp.float32)


def _vq_kernel(w_ref, flat_t_ref, out_ref, tile_v_ref, tile_i_ref, acc_v_ref):
    k = pl.program_id(0)
    j = jax.lax.rem(k, STEPS_PER_BLOCK)

    flat_t = flat_t_ref[...]
    w = w_ref[...]

    a = jnp.sum(flat_t * flat_t, axis=0, keepdims=True)
    c = jnp.sum(w * w, axis=1, keepdims=True)

    zq = _bf16_round(flat_t)
    dims = (((1,), (0,)), ((), ()))
    b = jax.lax.dot_general(w, zq, dims, preferred_element_type=jnp.float32)

    d = (a - 2.0 * b) + c

    loc_min = jnp.min(d, axis=0, keepdims=True)
    mask = d == loc_min
    iota = jax.lax.broadcasted_iota(jnp.int32, d.shape, 0) + k * SUB
    loc_idx = jnp.min(jnp.where(mask, iota, N_CODES), axis=0, keepdims=True)

    @pl.when(j == 0)
    def _start_block():
        tile_v_ref[...] = loc_min
        tile_i_ref[...] = loc_idx

    @pl.when(j != 0)
    def _merge_block():
        upd = loc_min < tile_v_ref[...]
        tile_i_ref[...] = jnp.where(upd, loc_idx, tile_i_ref[...])
        tile_v_ref[...] = jnp.where(upd, loc_min, tile_v_ref[...])

    @pl.when(k == STEPS_PER_BLOCK - 1)
    def _first_block_done():
        acc_v_ref[...] = _bf16_round(tile_v_ref[...])
        out_ref[...] = tile_i_ref[...]

    @pl.when((j == STEPS_PER_BLOCK - 1) & (k > STEPS_PER_BLOCK - 1))
    def _block_done():
        upd = tile_v_ref[...] < acc_v_ref[...]
        out_ref[...] = jnp.where(upd, tile_i_ref[...], out_ref[...])
        acc_v_ref[...] = jnp.where(upd, _bf16_round(tile_v_ref[...]),
                                   acc_v_ref[...])


def kernel(z_e_x, W):
    B, T, d_ = z_e_x.shape
    flat_t = z_e_x.reshape(-1, d_).T

    out = pl.pallas_call(
        _vq_kernel,
        grid=(N_CODES // SUB,),
        in_specs=[
            pl.BlockSpec((SUB, D), lambda k: (k, 0)),
            pl.BlockSpec((D, N_TOK), lambda k: (0, 0)),
        ],
        out_specs=pl.BlockSpec((1, N_TOK), lambda k: (0, 0)),
        out_shape=jax.ShapeDtypeStruct((1, N_TOK), jnp.int32),
        scratch_shapes=[
            pltpu.VMEM((1, N_TOK), jnp.float32),
            pltpu.VMEM((1, N_TOK), jnp.int32),
            pltpu.VMEM((1, N_TOK), jnp.float32),
        ],
        compiler_params=pltpu.CompilerParams(
            dimension_semantics=("arbitrary",)),
    )(W, flat_t)
    return out.reshape(B, T)


# speedup vs baseline: 1.4834x; 1.4571x over previous
import jax
import jax.numpy as jnp
from jax.experimental import pallas as pl
from jax.experimental.pallas import tpu as pltpu

N_CODES = 8192
N_TOK = 8192
D = 32
SUB = 256
BLOCK = 2048
STEPS_PER_BLOCK = BLOCK // SUB
ROWS = SUB // 8


def _bf16_round(x):
    return x.astype(jnp.bfloat16).astype(jnp.float32)


def _vq_kernel(w_ref, flat_t_ref, out_ref,
               a_ref, zq_ref, tile_v_ref, tile_i_ref, acc_v_ref):
    k = pl.program_id(0)
    j = jax.lax.rem(k, STEPS_PER_BLOCK)

    @pl.when(k == 0)
    def _prep():
        ft = flat_t_ref[...]
        a_ref[...] = jnp.sum(ft * ft, axis=0, keepdims=True)
        zq_ref[...] = _bf16_round(ft)

    w = w_ref[...]
    c = jnp.sum(w * w, axis=1, keepdims=True)
    dims = (((1,), (0,)), ((), ()))
    b2 = jax.lax.dot_general(w + w, zq_ref[...], dims,
                             preferred_element_type=jnp.float32)
    d = (a_ref[...] - b2) + c

    vs = [d[8 * i:8 * (i + 1), :] for i in range(ROWS)]
    ridx = [jnp.full((8, N_TOK), i, jnp.int32) for i in range(ROWS)]
    while len(vs) > 1:
        nv, ni = [], []
        for p in range(0, len(vs), 2):
            va, vb = vs[p], vs[p + 1]
            ia, ib = ridx[p], ridx[p + 1]
            t = vb < va
            nv.append(jnp.where(t, vb, va))
            ni.append(jnp.where(t, ib, ia))
        vs, ridx = nv, ni
    v8, r8 = vs[0], ridx[0]
    srow = jax.lax.broadcasted_iota(jnp.int32, (8, N_TOK), 0)
    code8 = r8 * 8 + srow
    loc_min = jnp.min(v8, axis=0, keepdims=True)
    m = v8 == loc_min
    loc_idx = jnp.min(jnp.where(m, code8, N_CODES), axis=0,
                      keepdims=True) + k * SUB

    @pl.when(j == 0)
    def _start_block():
        tile_v_ref[...] = loc_min
        tile_i_ref[...] = loc_idx

    @pl.when(j != 0)
    def _merge_block():
        upd = loc_min < tile_v_ref[...]
        tile_i_ref[...] = jnp.where(upd, loc_idx, tile_i_ref[...])
        tile_v_ref[...] = jnp.where(upd, loc_min, tile_v_ref[...])

    @pl.when(k == STEPS_PER_BLOCK - 1)
    def _first_block_done():
        acc_v_ref[...] = _bf16_round(tile_v_ref[...])
        out_ref[...] = tile_i_ref[...]

    @pl.when((j == STEPS_PER_BLOCK - 1) & (k > STEPS_PER_BLOCK - 1))
    def _block_done():
        upd = tile_v_ref[...] < acc_v_ref[...]
        out_ref[...] = jnp.where(upd, tile_i_ref[...], out_ref[...])
        acc_v_ref[...] = jnp.where(upd, _bf16_round(tile_v_ref[...]),
                                   acc_v_ref[...])


def kernel(z_e_x, W):
    B, T, d_ = z_e_x.shape
    flat_t = z_e_x.reshape(-1, d_).T

    out = pl.pallas_call(
        _vq_kernel,
        grid=(N_CODES // SUB,),
        in_specs=[
            pl.BlockSpec((SUB, D), lambda k: (k, 0)),
            pl.BlockSpec((D, N_TOK), lambda k: (0, 0)),
        ],
        out_specs=pl.BlockSpec((1, N_TOK), lambda k: (0, 0)),
        out_shape=jax.ShapeDtypeStruct((1, N_TOK), jnp.int32),
        scratch_shapes=[
            pltpu.VMEM((1, N_TOK), jnp.float32),
            pltpu.VMEM((D, N_TOK), jnp.float32),
            pltpu.VMEM((1, N_TOK), jnp.float32),
            pltpu.VMEM((1, N_TOK), jnp.int32),
            pltpu.VMEM((1, N_TOK), jnp.float32),
        ],
        compiler_params=pltpu.CompilerParams(
            dimension_semantics=("arbitrary",)),
    )(W, flat_t)
    return out.reshape(B, T)


# speedup vs baseline: 1.6630x; 1.1211x over previous
import jax
import jax.numpy as jnp
from jax.experimental import pallas as pl
from jax.experimental.pallas import tpu as pltpu

N_CODES = 8192
N_TOK = 8192
D = 32
SUB = 512
BLOCK = 2048
STEPS_PER_BLOCK = BLOCK // SUB
ROWS = SUB // 8


def _bf16_round(x):
    return x.astype(jnp.bfloat16).astype(jnp.float32)


def _vq_kernel(w_ref, flat_t_ref, out_ref,
               a_ref, zq_ref, tile_v_ref, tile_i_ref, acc_v_ref):
    k = pl.program_id(0)
    j = jax.lax.rem(k, STEPS_PER_BLOCK)

    @pl.when(k == 0)
    def _prep():
        ft = flat_t_ref[...]
        a_ref[...] = jnp.sum(ft * ft, axis=0, keepdims=True)
        zq_ref[...] = _bf16_round(ft)

    w = w_ref[...]
    c = jnp.sum(w * w, axis=1, keepdims=True)
    dims = (((1,), (0,)), ((), ()))
    b2 = jax.lax.dot_general(w + w, zq_ref[...], dims,
                             preferred_element_type=jnp.float32)
    d = (a_ref[...] - b2) + c

    vs = [d[8 * i:8 * (i + 1), :] for i in range(ROWS)]
    ridx = [jnp.full((8, N_TOK), i, jnp.int32) for i in range(ROWS)]
    while len(vs) > 1:
        nv, ni = [], []
        for p in range(0, len(vs), 2):
            va, vb = vs[p], vs[p + 1]
            ia, ib = ridx[p], ridx[p + 1]
            t = vb < va
            nv.append(jnp.where(t, vb, va))
            ni.append(jnp.where(t, ib, ia))
        vs, ridx = nv, ni
    v8, r8 = vs[0], ridx[0]
    srow = jax.lax.broadcasted_iota(jnp.int32, (8, N_TOK), 0)
    code8 = r8 * 8 + srow
    loc_min = jnp.min(v8, axis=0, keepdims=True)
    m = v8 == loc_min
    loc_idx = jnp.min(jnp.where(m, code8, N_CODES), axis=0,
                      keepdims=True) + k * SUB

    @pl.when(j == 0)
    def _start_block():
        tile_v_ref[...] = loc_min
        tile_i_ref[...] = loc_idx

    @pl.when(j != 0)
    def _merge_block():
        upd = loc_min < tile_v_ref[...]
        tile_i_ref[...] = jnp.where(upd, loc_idx, tile_i_ref[...])
        tile_v_ref[...] = jnp.where(upd, loc_min, tile_v_ref[...])

    @pl.when(k == STEPS_PER_BLOCK - 1)
    def _first_block_done():
        acc_v_ref[...] = _bf16_round(tile_v_ref[...])
        out_ref[...] = tile_i_ref[...]

    @pl.when((j == STEPS_PER_BLOCK - 1) & (k > STEPS_PER_BLOCK - 1))
    def _block_done():
        upd = tile_v_ref[...] < acc_v_ref[...]
        out_ref[...] = jnp.where(upd, tile_i_ref[...], out_ref[...])
        acc_v_ref[...] = jnp.where(upd, _bf16_round(tile_v_ref[...]),
                                   acc_v_ref[...])


def kernel(z_e_x, W):
    B, T, d_ = z_e_x.shape
    flat_t = z_e_x.reshape(-1, d_).T

    out = pl.pallas_call(
        _vq_kernel,
        grid=(N_CODES // SUB,),
        in_specs=[
            pl.BlockSpec((SUB, D), lambda k: (k, 0)),
            pl.BlockSpec((D, N_TOK), lambda k: (0, 0)),
        ],
        out_specs=pl.BlockSpec((1, N_TOK), lambda k: (0, 0)),
        out_shape=jax.ShapeDtypeStruct((1, N_TOK), jnp.int32),
        scratch_shapes=[
            pltpu.VMEM((1, N_TOK), jnp.float32),
            pltpu.VMEM((D, N_TOK), jnp.float32),
            pltpu.VMEM((1, N_TOK), jnp.float32),
            pltpu.VMEM((1, N_TOK), jnp.int32),
            pltpu.VMEM((1, N_TOK), jnp.float32),
        ],
        compiler_params=pltpu.CompilerParams(
            dimension_semantics=("arbitrary",)),
    )(W, flat_t)
    return out.reshape(B, T)


# speedup vs baseline: 1.7869x; 1.0745x over previous
import jax
import jax.numpy as jnp
from jax.experimental import pallas as pl
from jax.experimental.pallas import tpu as pltpu

N_CODES = 8192
N_TOK = 8192
D = 32
SUB = 1024
BLOCK = 2048
STEPS_PER_BLOCK = BLOCK // SUB
ROWS = SUB // 8


def _bf16_round(x):
    return x.astype(jnp.bfloat16).astype(jnp.float32)


def _vq_kernel(w_ref, flat_t_ref, out_ref,
               a_ref, zq_ref, tile_v_ref, tile_i_ref, acc_v_ref):
    k = pl.program_id(0)
    j = jax.lax.rem(k, STEPS_PER_BLOCK)

    @pl.when(k == 0)
    def _prep():
        ft = flat_t_ref[...]
        a_ref[...] = jnp.sum(ft * ft, axis=0, keepdims=True)
        zq_ref[...] = _bf16_round(ft)

    w = w_ref[...]
    c = jnp.sum(w * w, axis=1, keepdims=True)
    dims = (((1,), (0,)), ((), ()))
    b2 = jax.lax.dot_general(w + w, zq_ref[...], dims,
                             preferred_element_type=jnp.float32)
    d = (a_ref[...] - b2) + c

    vs = [d[8 * i:8 * (i + 1), :] for i in range(ROWS)]
    ridx = [jnp.full((8, N_TOK), i, jnp.int32) for i in range(ROWS)]
    while len(vs) > 1:
        nv, ni = [], []
        for p in range(0, len(vs), 2):
            va, vb = vs[p], vs[p + 1]
            ia, ib = ridx[p], ridx[p + 1]
            t = vb < va
            nv.append(jnp.where(t, vb, va))
            ni.append(jnp.where(t, ib, ia))
        vs, ridx = nv, ni
    v8, r8 = vs[0], ridx[0]
    srow = jax.lax.broadcasted_iota(jnp.int32, (8, N_TOK), 0)
    code8 = r8 * 8 + srow
    loc_min = jnp.min(v8, axis=0, keepdims=True)
    m = v8 == loc_min
    loc_idx = jnp.min(jnp.where(m, code8, N_CODES), axis=0,
                      keepdims=True) + k * SUB

    @pl.when(j == 0)
    def _start_block():
        tile_v_ref[...] = loc_min
        tile_i_ref[...] = loc_idx

    @pl.when(j != 0)
    def _merge_block():
        upd = loc_min < tile_v_ref[...]
        tile_i_ref[...] = jnp.where(upd, loc_idx, tile_i_ref[...])
        tile_v_ref[...] = jnp.where(upd, loc_min, tile_v_ref[...])

    @pl.when(k == STEPS_PER_BLOCK - 1)
    def _first_block_done():
        acc_v_ref[...] = _bf16_round(tile_v_ref[...])
        out_ref[...] = tile_i_ref[...]

    @pl.when((j == STEPS_PER_BLOCK - 1) & (k > STEPS_PER_BLOCK - 1))
    def _block_done():
        upd = tile_v_ref[...] < acc_v_ref[...]
        out_ref[...] = jnp.where(upd, tile_i_ref[...], out_ref[...])
        acc_v_ref[...] = jnp.where(upd, _bf16_round(tile_v_ref[...]),
                                   acc_v_ref[...])


def kernel(z_e_x, W):
    B, T, d_ = z_e_x.shape
    flat_t = z_e_x.reshape(-1, d_).T

    out = pl.pallas_call(
        _vq_kernel,
        grid=(N_CODES // SUB,),
        in_specs=[
            pl.BlockSpec((SUB, D), lambda k: (k, 0)),
            pl.BlockSpec((D, N_TOK), lambda k: (0, 0)),
        ],
        out_specs=pl.BlockSpec((1, N_TOK), lambda k: (0, 0)),
        out_shape=jax.ShapeDtypeStruct((1, N_TOK), jnp.int32),
        scratch_shapes=[
            pltpu.VMEM((1, N_TOK), jnp.float32),
            pltpu.VMEM((D, N_TOK), jnp.float32),
            pltpu.VMEM((1, N_TOK), jnp.float32),
            pltpu.VMEM((1, N_TOK), jnp.int32),
            pltpu.VMEM((1, N_TOK), jnp.float32),
        ],
        compiler_params=pltpu.CompilerParams(
            dimension_semantics=("arbitrary",)),
    )(W, flat_t)
    return out.reshape(B, T)


# speedup vs baseline: 1.8324x; 1.0255x over previous
import jax
import jax.numpy as jnp
from jax.experimental import pallas as pl
from jax.experimental.pallas import tpu as pltpu

N_CODES = 8192
N_TOK = 8192
D = 32
SUB = 2048
BLOCK = 2048
STEPS_PER_BLOCK = BLOCK // SUB
ROWS = SUB // 8


def _bf16_round(x):
    return x.astype(jnp.bfloat16).astype(jnp.float32)


def _vq_kernel(w_ref, flat_t_ref, out_ref,
               a_ref, zq_ref, tile_v_ref, tile_i_ref, acc_v_ref):
    k = pl.program_id(0)
    j = jax.lax.rem(k, STEPS_PER_BLOCK)

    @pl.when(k == 0)
    def _prep():
        ft = flat_t_ref[...]
        a_ref[...] = jnp.sum(ft * ft, axis=0, keepdims=True)
        zq_ref[...] = _bf16_round(ft)

    w = w_ref[...]
    c = jnp.sum(w * w, axis=1, keepdims=True)
    dims = (((1,), (0,)), ((), ()))
    b2 = jax.lax.dot_general(w + w, zq_ref[...], dims,
                             preferred_element_type=jnp.float32)
    d = (a_ref[...] - b2) + c

    vs = [d[8 * i:8 * (i + 1), :] for i in range(ROWS)]
    ridx = [jnp.full((8, N_TOK), i, jnp.int32) for i in range(ROWS)]
    while len(vs) > 1:
        nv, ni = [], []
        for p in range(0, len(vs), 2):
            va, vb = vs[p], vs[p + 1]
            ia, ib = ridx[p], ridx[p + 1]
            t = vb < va
            nv.append(jnp.where(t, vb, va))
            ni.append(jnp.where(t, ib, ia))
        vs, ridx = nv, ni
    v8, r8 = vs[0], ridx[0]
    srow = jax.lax.broadcasted_iota(jnp.int32, (8, N_TOK), 0)
    code8 = r8 * 8 + srow
    loc_min = jnp.min(v8, axis=0, keepdims=True)
    m = v8 == loc_min
    loc_idx = jnp.min(jnp.where(m, code8, N_CODES), axis=0,
                      keepdims=True) + k * SUB

    @pl.when(j == 0)
    def _start_block():
        tile_v_ref[...] = loc_min
        tile_i_ref[...] = loc_idx

    @pl.when(j != 0)
    def _merge_block():
        upd = loc_min < tile_v_ref[...]
        tile_i_ref[...] = jnp.where(upd, loc_idx, tile_i_ref[...])
        tile_v_ref[...] = jnp.where(upd, loc_min, tile_v_ref[...])

    @pl.when(k == STEPS_PER_BLOCK - 1)
    def _first_block_done():
        acc_v_ref[...] = _bf16_round(tile_v_ref[...])
        out_ref[...] = tile_i_ref[...]

    @pl.when((j == STEPS_PER_BLOCK - 1) & (k > STEPS_PER_BLOCK - 1))
    def _block_done():
        upd = tile_v_ref[...] < acc_v_ref[...]
        out_ref[...] = jnp.where(upd, tile_i_ref[...], out_ref[...])
        acc_v_ref[...] = jnp.where(upd, _bf16_round(tile_v_ref[...]),
                                   acc_v_ref[...])


def kernel(z_e_x, W):
    B, T, d_ = z_e_x.shape
    flat_t = z_e_x.reshape(-1, d_).T

    out = pl.pallas_call(
        _vq_kernel,
        grid=(N_CODES // SUB,),
        in_specs=[
            pl.BlockSpec((SUB, D), lambda k: (k, 0)),
            pl.BlockSpec((D, N_TOK), lambda k: (0, 0)),
        ],
        out_specs=pl.BlockSpec((1, N_TOK), lambda k: (0, 0)),
        out_shape=jax.ShapeDtypeStruct((1, N_TOK), jnp.int32),
        scratch_shapes=[
            pltpu.VMEM((1, N_TOK), jnp.float32),
            pltpu.VMEM((D, N_TOK), jnp.float32),
            pltpu.VMEM((1, N_TOK), jnp.float32),
            pltpu.VMEM((1, N_TOK), jnp.int32),
            pltpu.VMEM((1, N_TOK), jnp.float32),
        ],
        compiler_params=pltpu.CompilerParams(
            dimension_semantics=("arbitrary",)),
    )(W, flat_t)
    return out.reshape(B, T)
